# Initial kernel scaffold; baseline (speedup 1.0000x reference)
import jax
import jax.numpy as jnp
from jax.experimental import pallas as pl


def kernel(embed_id, weight):
    raise NotImplementedError("write your pallas kernel here")


# speedup vs baseline: 1.4260x; 1.4260x over previous
import jax
import jax.numpy as jnp
from jax import lax
from jax.experimental import pallas as pl
from jax.experimental.pallas import tpu as pltpu
from jax.experimental.pallas import tpu_sc as plsc

NUM_CORES = 2
NUM_SUBCORES = 16
NW = NUM_CORES * NUM_SUBCORES

CHUNK = 128
D = 32


def _gather_body(idx_hbm, table_hbm, out_hbm, idx_v, buf0, buf1, sem0, sem1):
    wid = lax.axis_index("s") * NUM_CORES + lax.axis_index("c")
    n_chunks = idx_hbm.shape[1]

    pltpu.sync_copy(idx_hbm.at[wid], idx_v)

    cp0 = pltpu.async_copy(table_hbm.at[idx_v.at[0]], buf0, sem0)
    cp1 = pltpu.async_copy(table_hbm.at[idx_v.at[1]], buf1, sem1)

    @pl.loop(0, n_chunks - 2, step=2)
    def _(base):
        pltpu.make_async_copy(table_hbm.at[idx_v.at[base]], buf0, sem0).wait()
        pltpu.sync_copy(buf0, out_hbm.at[wid, base])
        pltpu.async_copy(table_hbm.at[idx_v.at[base + 2]], buf0, sem0)
        pltpu.make_async_copy(table_hbm.at[idx_v.at[base + 1]], buf1, sem1).wait()
        pltpu.sync_copy(buf1, out_hbm.at[wid, base + 1])
        pltpu.async_copy(table_hbm.at[idx_v.at[base + 3]], buf1, sem1)

    cp0.wait()
    pltpu.sync_copy(buf0, out_hbm.at[wid, n_chunks - 2])
    cp1.wait()
    pltpu.sync_copy(buf1, out_hbm.at[wid, n_chunks - 1])


def kernel(embed_id, weight):
    batch, hist = embed_id.shape
    total = batch * hist
    assert total % (NW * CHUNK) == 0
    n_chunks = total // (NW * CHUNK)

    idx3 = embed_id.astype(jnp.int32).reshape(NW, n_chunks, CHUNK)

    mesh = plsc.VectorSubcoreMesh(
        core_axis_name="c", subcore_axis_name="s",
        num_cores=NUM_CORES, num_subcores=NUM_SUBCORES,
    )
    run = pl.kernel(
        _gather_body,
        out_type=jax.ShapeDtypeStruct((NW, n_chunks, CHUNK, D), jnp.float32),
        mesh=mesh,
        compiler_params=pltpu.CompilerParams(use_tc_tiling_on_sc=False),
        scratch_types=[
            pltpu.VMEM((n_chunks, CHUNK), jnp.int32),
            pltpu.VMEM((CHUNK, D), jnp.float32),
            pltpu.VMEM((CHUNK, D), jnp.float32),
            pltpu.SemaphoreType.DMA,
            pltpu.SemaphoreType.DMA,
        ],
    )
    out = run(idx3, weight)
    return out.reshape(batch, hist, D)


# speedup vs baseline: 1.4996x; 1.0516x over previous
import jax
import jax.numpy as jnp
from jax import lax
from jax.experimental import pallas as pl
from jax.experimental.pallas import tpu as pltpu
from jax.experimental.pallas import tpu_sc as plsc

NUM_CORES = 2
NUM_SUBCORES = 16
NW = NUM_CORES * NUM_SUBCORES

CHUNK = 512
D = 32


def _gather_body(idx_hbm, table_hbm, out_hbm, idx_v, buf0, buf1, sem0, sem1):
    wid = lax.axis_index("s") * NUM_CORES + lax.axis_index("c")
    n_chunks = idx_hbm.shape[1]

    pltpu.sync_copy(idx_hbm.at[wid], idx_v)

    cp0 = pltpu.async_copy(table_hbm.at[idx_v.at[0]], buf0, sem0)
    cp1 = pltpu.async_copy(table_hbm.at[idx_v.at[1]], buf1, sem1)

    @pl.loop(0, n_chunks - 2, step=2)
    def _(base):
        pltpu.make_async_copy(table_hbm.at[idx_v.at[base]], buf0, sem0).wait()
        pltpu.sync_copy(buf0, out_hbm.at[wid, base])
        pltpu.async_copy(table_hbm.at[idx_v.at[base + 2]], buf0, sem0)
        pltpu.make_async_copy(table_hbm.at[idx_v.at[base + 1]], buf1, sem1).wait()
        pltpu.sync_copy(buf1, out_hbm.at[wid, base + 1])
        pltpu.async_copy(table_hbm.at[idx_v.at[base + 3]], buf1, sem1)

    cp0.wait()
    pltpu.sync_copy(buf0, out_hbm.at[wid, n_chunks - 2])
    cp1.wait()
    pltpu.sync_copy(buf1, out_hbm.at[wid, n_chunks - 1])


def kernel(embed_id, weight):
    batch, hist = embed_id.shape
    total = batch * hist
    assert total % (NW * CHUNK) == 0
    n_chunks = total // (NW * CHUNK)

    idx3 = embed_id.astype(jnp.int32).reshape(NW, n_chunks, CHUNK)

    mesh = plsc.VectorSubcoreMesh(
        core_axis_name="c", subcore_axis_name="s",
        num_cores=NUM_CORES, num_subcores=NUM_SUBCORES,
    )
    run = pl.kernel(
        _gather_body,
        out_type=jax.ShapeDtypeStruct((NW, n_chunks, CHUNK, D), jnp.float32),
        mesh=mesh,
        compiler_params=pltpu.CompilerParams(use_tc_tiling_on_sc=False),
        scratch_types=[
            pltpu.VMEM((n_chunks, CHUNK), jnp.int32),
            pltpu.VMEM((CHUNK, D), jnp.float32),
            pltpu.VMEM((CHUNK, D), jnp.float32),
            pltpu.SemaphoreType.DMA,
            pltpu.SemaphoreType.DMA,
        ],
    )
    out = run(idx3, weight)
    return out.reshape(batch, hist, D)
